# baseline (speedup 1.0000x reference)
import dataclasses
import functools

import jax
import jax.numpy as jnp
from jax.experimental import pallas as pl
from jax.experimental.pallas import tpu as pltpu
from jax.experimental.pallas import tpu_sc as plsc

_ROWS = 160
_K = 64
_LANES = 16


def _sc_run(n_rows, d_hbm, r_hbm, s_hbm, o_hbm, rinv_v):
    def body(dist_v, rho_v, sig_v, out_v):
        @pl.loop(0, _ROWS // _LANES)
        def _(g):
            sl = pl.ds(g * _LANES, _LANES)
            rinv_v[sl] = 1.0 / sig_v[sl]

        @pl.loop(0, _ROWS)
        def _(r):
            idx = jnp.full((_LANES,), r, jnp.int32)
            rho_b = plsc.load_gather(rho_v, [idx])
            rinv_b = plsc.load_gather(rinv_v, [idx])
            for c in range(_K // _LANES):
                sl = pl.ds(r * _K + c * _LANES, _LANES)
                x = dist_v[sl]
                out_v[sl] = jnp.minimum(jnp.exp((rho_b - x) * rinv_b), 1.0)

    pltpu.emit_pipeline(
        body,
        grid=(n_rows // _ROWS,),
        in_specs=[
            pl.BlockSpec((_ROWS * _K,), lambda i: (i,)),
            pl.BlockSpec((_ROWS,), lambda i: (i,)),
            pl.BlockSpec((_ROWS,), lambda i: (i,)),
        ],
        out_specs=[pl.BlockSpec((_ROWS * _K,), lambda i: (i,))],
        core_axis_name=("c", "s"),
        dimension_semantics=(pltpu.PARALLEL,),
    )(d_hbm, r_hbm, s_hbm, o_hbm)


def kernel(distances, rho, sigma):
    n, k = distances.shape
    mesh = plsc.VectorSubcoreMesh(core_axis_name="c", subcore_axis_name="s")
    cp = pltpu.CompilerParams()
    if "needs_layout_passes" in pltpu.CompilerParams.__dataclass_fields__:
        cp = dataclasses.replace(cp, needs_layout_passes=False)
    run = functools.partial(
        pl.kernel,
        out_type=jax.ShapeDtypeStruct((n * k,), jnp.float32),
        mesh=mesh,
        scratch_types=[pltpu.VMEM((_ROWS,), jnp.float32)],
        compiler_params=cp,
    )(functools.partial(_sc_run, n))
    out = run(distances.reshape(-1), rho, sigma)
    return out.reshape(n, k)


# speedup vs baseline: 1.9612x; 1.9612x over previous
import dataclasses
import functools

import jax
import jax.numpy as jnp
from jax.experimental import pallas as pl
from jax.experimental.pallas import tpu as pltpu
from jax.experimental.pallas import tpu_sc as plsc

_ROWS = 160
_K = 64
_LANES = 16


def _sc_run(n_rows, d_hbm, r_hbm, s_hbm, o_hbm, rinv_v):
    def body(dist_v, rho_v, sig_v, out_v):
        @functools.partial(plsc.parallel_loop, 0, _ROWS // _LANES, unroll=4)
        def _(g):
            sl = pl.ds(g * _LANES, _LANES)
            rinv_v[sl] = 1.0 / sig_v[sl]

        @functools.partial(plsc.parallel_loop, 0, _ROWS, unroll=4)
        def _(r):
            idx = jnp.full((_LANES,), r, jnp.int32)
            rho_b = plsc.load_gather(rho_v, [idx])
            rinv_b = plsc.load_gather(rinv_v, [idx])
            for c in range(_K // _LANES):
                sl = pl.ds(r * _K + c * _LANES, _LANES)
                x = dist_v[sl]
                out_v[sl] = jnp.minimum(jnp.exp((rho_b - x) * rinv_b), 1.0)

    pltpu.emit_pipeline(
        body,
        grid=(n_rows // _ROWS,),
        in_specs=[
            pl.BlockSpec((_ROWS * _K,), lambda i: (i,)),
            pl.BlockSpec((_ROWS,), lambda i: (i,)),
            pl.BlockSpec((_ROWS,), lambda i: (i,)),
        ],
        out_specs=[pl.BlockSpec((_ROWS * _K,), lambda i: (i,))],
        core_axis_name=("c", "s"),
        dimension_semantics=(pltpu.PARALLEL,),
    )(d_hbm, r_hbm, s_hbm, o_hbm)


def kernel(distances, rho, sigma):
    n, k = distances.shape
    mesh = plsc.VectorSubcoreMesh(core_axis_name="c", subcore_axis_name="s")
    cp = pltpu.CompilerParams()
    if "needs_layout_passes" in pltpu.CompilerParams.__dataclass_fields__:
        cp = dataclasses.replace(cp, needs_layout_passes=False)
    run = functools.partial(
        pl.kernel,
        out_type=jax.ShapeDtypeStruct((n * k,), jnp.float32),
        mesh=mesh,
        scratch_types=[pltpu.VMEM((_ROWS,), jnp.float32)],
        compiler_params=cp,
    )(functools.partial(_sc_run, n))
    out = run(distances.reshape(-1), rho, sigma)
    return out.reshape(n, k)
